# baseline (speedup 1.0000x reference)
import dataclasses
import functools

import jax
import jax.numpy as jnp
from jax import lax
from jax.experimental import pallas as pl
from jax.experimental.pallas import tpu as pltpu
from jax.experimental.pallas import tpu_sc as plsc

N = 10000
E = 320000
D_IN = 128
D_HID = 256
D_OUT = 128
DH = 128

NC = 2
NS = 16
NW = NC * NS

CH = 128
E_ROWS = 2560
E_PAD = E_ROWS * CH - E
T1 = E_ROWS // NS
T2 = E_ROWS // NW
NBUF = 2
G = 16

RB = 80
NBLK = N // RB
N_P = 10008

BR = 1000
f32 = jnp.float32


def _mesh():
    return plsc.VectorSubcoreMesh(core_axis_name="c", subcore_axis_name="s")


def _sc_params():
    cp = pltpu.CompilerParams()
    if "needs_layout_passes" in pltpu.CompilerParams.__dataclass_fields__:
        cp = dataclasses.replace(cp, needs_layout_passes=False)
    return cp


def _sc_hist(dst32):
    @functools.partial(
        pl.kernel,
        out_type=jax.ShapeDtypeStruct((NW, N), f32),
        mesh=_mesh(),
        scratch_types=[
            pltpu.VMEM((N,), f32),
            pltpu.VMEM((E // NW,), jnp.int32),
            pltpu.SemaphoreType.DMA,
        ],
        compiler_params=_sc_params(),
    )
    def k(dst_hbm, out_hbm, hist_v, idx_v, sem):
        c = lax.axis_index("c")
        s = lax.axis_index("s")
        wid = s * NC + c
        zero16 = jnp.zeros((16,), f32)
        one16 = jnp.full((16,), 1.0, f32)

        cp = pltpu.async_copy(dst_hbm.at[wid], idx_v, sem)

        @pl.loop(0, N // 16)
        def _(i):
            hist_v[pl.ds(i * 16, 16)] = zero16

        cp.wait()

        @pl.loop(0, (E // NW) // 16)
        def _(i):
            idx = idx_v[pl.ds(i * 16, 16)]
            plsc.addupdate_scatter(hist_v, [idx], one16)

        pltpu.sync_copy(hist_v, out_hbm.at[wid])

    return k(dst32)


def _sc_agg(g, src2, dst2, feature_split):
    TCNT = T1 if feature_split else T2
    TG = TCNT // G

    @functools.partial(
        pl.kernel,
        out_type=jax.ShapeDtypeStruct((NC, N_P, DH), f32),
        mesh=_mesh(),
        scratch_types=[
            pltpu.VMEM_SHARED((N_P, DH), f32),
            pltpu.VMEM((G, CH), jnp.int32),
            pltpu.VMEM((G, CH), jnp.int32),
            pltpu.VMEM((NBUF, CH, DH), f32),
            [pltpu.SemaphoreType.DMA] * NBUF,
            [pltpu.SemaphoreType.DMA] * NBUF,
        ],
        compiler_params=_sc_params(),
    )
    def k(g_hbm, src_hbm, dst_hbm, out_hbm, acc_sh, src_idx, dst_idx,
          rows_v, gsem, ssem):
        c = lax.axis_index("c")
        s = lax.axis_index("s")

        gsrc = g_hbm.at[c] if feature_split else g_hbm
        lo = s * TCNT if feature_split else (c * NS + s) * TCNT

        if not feature_split:
            zero16 = jnp.zeros((16,), f32)

            @pl.when(c == 1)
            def _():
                @pl.loop(0, RB)
                def _(r):
                    for j in range(DH // 16):
                        rows_v[0, r, pl.ds(j * 16, 16)] = zero16

        for b in range(8):
            kk = b * NS + s
            base = kk * RB

            @pl.when(kk < NBLK)
            def _():
                if feature_split:
                    pltpu.sync_copy(g_hbm.at[c].at[pl.ds(base, RB)],
                                    rows_v.at[1].at[pl.ds(0, RB)])
                    pltpu.sync_copy(rows_v.at[1].at[pl.ds(0, RB)],
                                    acc_sh.at[pl.ds(base, RB)])
                else:
                    @pl.when(c == 0)
                    def _():
                        pltpu.sync_copy(g_hbm.at[pl.ds(base, RB)],
                                        rows_v.at[1].at[pl.ds(0, RB)])
                        pltpu.sync_copy(rows_v.at[1].at[pl.ds(0, RB)],
                                        acc_sh.at[pl.ds(base, RB)])

                    @pl.when(c == 1)
                    def _():
                        pltpu.sync_copy(rows_v.at[0].at[pl.ds(0, RB)],
                                        acc_sh.at[pl.ds(base, RB)])

        plsc.subcore_barrier()

        def issue_gather(b, j):
            pltpu.async_copy(gsrc.at[src_idx.at[j]], rows_v.at[b], gsem[b])

        def wait_gather(b):
            pltpu.make_async_copy(gsrc.at[src_idx.at[0]], rows_v.at[b],
                                  gsem[b]).wait()

        def issue_scatter(b, j):
            pltpu.async_copy(rows_v.at[b], acc_sh.at[dst_idx.at[j]],
                             ssem[b], add=True)

        def wait_scatter(b):
            pltpu.make_async_copy(rows_v.at[b], acc_sh.at[dst_idx.at[0]],
                                  ssem[b]).wait()

        @pl.loop(0, TG)
        def _(gg):
            row0 = lo + gg * G
            pltpu.sync_copy(src_hbm.at[pl.ds(row0, G)], src_idx)
            pltpu.sync_copy(dst_hbm.at[pl.ds(row0, G)], dst_idx)
            for b in range(NBUF):
                issue_gather(b, b)

            @pl.loop(0, G // NBUF - 1)
            def _(r):
                for b in range(NBUF):
                    wait_gather(b)
                    issue_scatter(b, r * NBUF + b)
                for b in range(NBUF):
                    wait_scatter(b)
                    issue_gather(b, (r + 1) * NBUF + b)

            for b in range(NBUF):
                wait_gather(b)
                issue_scatter(b, G - NBUF + b)
            for b in range(NBUF):
                wait_scatter(b)

        plsc.subcore_barrier()

        for b in range(8):
            kk = b * NS + s
            base = kk * RB

            @pl.when(kk < NBLK)
            def _():
                pltpu.sync_copy(acc_sh.at[pl.ds(base, RB)],
                                rows_v.at[0].at[pl.ds(0, RB)])
                pltpu.sync_copy(rows_v.at[0].at[pl.ds(0, RB)],
                                out_hbm.at[c].at[pl.ds(base, RB)])

    return k(g, src2, dst2)


def _tc1(x, W1, histT):
    def body(x_ref, w_ref, h_ref, g_ref, dis_ref):
        cnt = jnp.sum(h_ref[...], axis=1, keepdims=True)
        dis = lax.rsqrt(cnt + 1.0)
        h1 = jnp.dot(x_ref[...], w_ref[...], preferred_element_type=f32)
        gg = h1 * dis
        g_ref[0] = gg[:, :DH]
        g_ref[1] = gg[:, DH:]
        dis_ref[...] = dis

    return pl.pallas_call(
        body,
        grid=(N // BR,),
        in_specs=[
            pl.BlockSpec((BR, D_IN), lambda i: (i, 0)),
            pl.BlockSpec((D_IN, D_HID), lambda i: (0, 0)),
            pl.BlockSpec((BR, NW), lambda i: (i, 0)),
        ],
        out_specs=[
            pl.BlockSpec((2, BR, DH), lambda i: (0, i, 0)),
            pl.BlockSpec((BR, 1), lambda i: (i, 0)),
        ],
        out_shape=[
            jax.ShapeDtypeStruct((NC, N, DH), f32),
            jax.ShapeDtypeStruct((N, 1), f32),
        ],
    )(x, W1, histT)


def _tc2(agg1, dis, b1, W2):
    def body(a_ref, dis_ref, b_ref, w_ref, o_ref):
        a = jnp.concatenate([a_ref[0], a_ref[1]], axis=1)
        d = dis_ref[...]
        z = jnp.maximum(a * d + b_ref[...], 0.0)
        h2 = jnp.dot(z, w_ref[...], preferred_element_type=f32)
        o_ref[...] = h2 * d

    return pl.pallas_call(
        body,
        grid=(N // BR,),
        in_specs=[
            pl.BlockSpec((2, BR, DH), lambda i: (0, i, 0)),
            pl.BlockSpec((BR, 1), lambda i: (i, 0)),
            pl.BlockSpec((1, D_HID), lambda i: (0, 0)),
            pl.BlockSpec((D_HID, D_OUT), lambda i: (0, 0)),
        ],
        out_specs=pl.BlockSpec((BR, D_OUT), lambda i: (i, 0)),
        out_shape=jax.ShapeDtypeStruct((N, D_OUT), f32),
    )(agg1, dis, b1, W2)


def _tc3(agg2, dis, b2):
    def body(a_ref, dis_ref, b_ref, o_ref):
        a = a_ref[0] + a_ref[1]
        o_ref[...] = jnp.maximum(a * dis_ref[...] + b_ref[...], 0.0)

    return pl.pallas_call(
        body,
        grid=(N // BR,),
        in_specs=[
            pl.BlockSpec((2, BR, DH), lambda i: (0, i, 0)),
            pl.BlockSpec((BR, 1), lambda i: (i, 0)),
            pl.BlockSpec((1, D_OUT), lambda i: (0, 0)),
        ],
        out_specs=pl.BlockSpec((BR, D_OUT), lambda i: (i, 0)),
        out_shape=jax.ShapeDtypeStruct((N, D_OUT), f32),
    )(agg2, dis, b2)


def kernel(x, edge_index, W1, b1, W2, b2):
    src = edge_index[0]
    dst = edge_index[1]
    pad_src = jnp.zeros((E_PAD,), jnp.int32)
    pad_dst = jnp.full((E_PAD,), N, jnp.int32)
    src2 = jnp.concatenate([src, pad_src]).reshape(E_ROWS, CH)
    dst2 = jnp.concatenate([dst, pad_dst]).reshape(E_ROWS, CH)
    dst32 = dst.reshape(NW, E // NW)

    hists = _sc_hist(dst32)
    histT = jnp.transpose(hists)

    g1, dis = _tc1(x, W1, histT)
    agg1 = _sc_agg(g1, src2, dst2, feature_split=True)
    g2 = _tc2(agg1, dis, b1.reshape(1, D_HID), W2)
    agg2 = _sc_agg(g2, src2, dst2, feature_split=False)
    out = _tc3(agg2, dis, b2.reshape(1, D_OUT))
    return out


# speedup vs baseline: 2.4474x; 2.4474x over previous
import dataclasses
import functools

import jax
import jax.numpy as jnp
from jax import lax
from jax.experimental import pallas as pl
from jax.experimental.pallas import tpu as pltpu
from jax.experimental.pallas import tpu_sc as plsc

N = 10000
E = 320000
D_IN = 128
D_HID = 256
D_OUT = 128
DH = 128

NC = 2
NS = 16
NW = NC * NS

CH = 128
E_ROWS = 2560
E_PAD = E_ROWS * CH - E
T1 = E_ROWS // NS
T2 = E_ROWS // NW
NBUF = 2
G = 16

RB = 80
NBLK = N // RB
JUNK = 512
N_P = N + JUNK

BR = 1000
f32 = jnp.float32


def _mesh():
    return plsc.VectorSubcoreMesh(core_axis_name="c", subcore_axis_name="s")


def _sc_params():
    cp = pltpu.CompilerParams()
    if "needs_layout_passes" in pltpu.CompilerParams.__dataclass_fields__:
        cp = dataclasses.replace(cp, needs_layout_passes=False)
    return cp


def _sc_hist(dst32):
    @functools.partial(
        pl.kernel,
        out_type=jax.ShapeDtypeStruct((NW, N), f32),
        mesh=_mesh(),
        scratch_types=[
            pltpu.VMEM((N,), f32),
            pltpu.VMEM((E // NW,), jnp.int32),
            pltpu.SemaphoreType.DMA,
        ],
        compiler_params=_sc_params(),
    )
    def k(dst_hbm, out_hbm, hist_v, idx_v, sem):
        c = lax.axis_index("c")
        s = lax.axis_index("s")
        wid = s * NC + c
        zero16 = jnp.zeros((16,), f32)
        one16 = jnp.full((16,), 1.0, f32)

        cp = pltpu.async_copy(dst_hbm.at[wid], idx_v, sem)

        @pl.loop(0, N // 16)
        def _(i):
            hist_v[pl.ds(i * 16, 16)] = zero16

        cp.wait()

        @pl.loop(0, (E // NW) // 16)
        def _(i):
            idx = idx_v[pl.ds(i * 16, 16)]
            plsc.addupdate_scatter(hist_v, [idx], one16)

        pltpu.sync_copy(hist_v, out_hbm.at[wid])

    return k(dst32)


def _sc_agg(g, src2, dst2, feature_split):
    TCNT = T1 if feature_split else T2
    TG = TCNT // G

    @functools.partial(
        pl.kernel,
        out_type=jax.ShapeDtypeStruct((NC, N_P, DH), f32),
        mesh=_mesh(),
        scratch_types=[
            pltpu.VMEM_SHARED((N_P, DH), f32),
            pltpu.VMEM((G, CH), jnp.int32),
            pltpu.VMEM((G, CH), jnp.int32),
            pltpu.VMEM((NBUF, CH, DH), f32),
            [pltpu.SemaphoreType.DMA] * NBUF,
            [pltpu.SemaphoreType.DMA] * NBUF,
        ],
        compiler_params=_sc_params(),
    )
    def k(g_hbm, src_hbm, dst_hbm, out_hbm, acc_sh, src_idx, dst_idx,
          rows_v, gsem, ssem):
        c = lax.axis_index("c")
        s = lax.axis_index("s")

        gsrc = g_hbm.at[c] if feature_split else g_hbm
        lo = s * TCNT if feature_split else (c * NS + s) * TCNT

        if not feature_split:
            zero16 = jnp.zeros((16,), f32)

            @pl.when(c == 1)
            def _():
                @pl.loop(0, RB)
                def _(r):
                    for j in range(DH // 16):
                        rows_v[0, r, pl.ds(j * 16, 16)] = zero16

        for b in range(8):
            kk = b * NS + s
            base = kk * RB

            @pl.when(kk < NBLK)
            def _():
                if feature_split:
                    pltpu.sync_copy(g_hbm.at[c].at[pl.ds(base, RB)],
                                    rows_v.at[1].at[pl.ds(0, RB)])
                    pltpu.sync_copy(rows_v.at[1].at[pl.ds(0, RB)],
                                    acc_sh.at[pl.ds(base, RB)])
                else:
                    @pl.when(c == 0)
                    def _():
                        pltpu.sync_copy(g_hbm.at[pl.ds(base, RB)],
                                        rows_v.at[1].at[pl.ds(0, RB)])
                        pltpu.sync_copy(rows_v.at[1].at[pl.ds(0, RB)],
                                        acc_sh.at[pl.ds(base, RB)])

                    @pl.when(c == 1)
                    def _():
                        pltpu.sync_copy(rows_v.at[0].at[pl.ds(0, RB)],
                                        acc_sh.at[pl.ds(base, RB)])

        plsc.subcore_barrier()

        def issue_gather(b, j):
            pltpu.async_copy(gsrc.at[src_idx.at[j]], rows_v.at[b], gsem[b])

        def wait_gather(b):
            pltpu.make_async_copy(gsrc.at[src_idx.at[0]], rows_v.at[b],
                                  gsem[b]).wait()

        def issue_scatter(b, j):
            pltpu.async_copy(rows_v.at[b], acc_sh.at[dst_idx.at[j]],
                             ssem[b], add=True)

        def wait_scatter(b):
            pltpu.make_async_copy(rows_v.at[b], acc_sh.at[dst_idx.at[0]],
                                  ssem[b]).wait()

        @pl.loop(0, TG)
        def _(gg):
            row0 = lo + gg * G
            pltpu.sync_copy(src_hbm.at[pl.ds(row0, G)], src_idx)
            pltpu.sync_copy(dst_hbm.at[pl.ds(row0, G)], dst_idx)
            for b in range(NBUF):
                issue_gather(b, b)

            @pl.loop(0, G // NBUF - 1)
            def _(r):
                for b in range(NBUF):
                    wait_gather(b)
                    issue_scatter(b, r * NBUF + b)
                for b in range(NBUF):
                    wait_scatter(b)
                    issue_gather(b, (r + 1) * NBUF + b)

            for b in range(NBUF):
                wait_gather(b)
                issue_scatter(b, G - NBUF + b)
            for b in range(NBUF):
                wait_scatter(b)

        plsc.subcore_barrier()

        for b in range(8):
            kk = b * NS + s
            base = kk * RB

            @pl.when(kk < NBLK)
            def _():
                pltpu.sync_copy(acc_sh.at[pl.ds(base, RB)],
                                rows_v.at[0].at[pl.ds(0, RB)])
                pltpu.sync_copy(rows_v.at[0].at[pl.ds(0, RB)],
                                out_hbm.at[c].at[pl.ds(base, RB)])

    return k(g, src2, dst2)


def _tc1(x, W1, histT):
    def body(x_ref, w_ref, h_ref, g_ref, dis_ref):
        cnt = jnp.sum(h_ref[...], axis=1, keepdims=True)
        dis = lax.rsqrt(cnt + 1.0)
        h1 = jnp.dot(x_ref[...], w_ref[...], preferred_element_type=f32)
        gg = h1 * dis
        g_ref[0] = gg[:, :DH]
        g_ref[1] = gg[:, DH:]
        dis_ref[...] = dis

    return pl.pallas_call(
        body,
        grid=(N // BR,),
        in_specs=[
            pl.BlockSpec((BR, D_IN), lambda i: (i, 0)),
            pl.BlockSpec((D_IN, D_HID), lambda i: (0, 0)),
            pl.BlockSpec((BR, NW), lambda i: (i, 0)),
        ],
        out_specs=[
            pl.BlockSpec((2, BR, DH), lambda i: (0, i, 0)),
            pl.BlockSpec((BR, 1), lambda i: (i, 0)),
        ],
        out_shape=[
            jax.ShapeDtypeStruct((NC, N, DH), f32),
            jax.ShapeDtypeStruct((N, 1), f32),
        ],
    )(x, W1, histT)


def _tc2(agg1, dis, b1, W2):
    def body(a_ref, dis_ref, b_ref, w_ref, o_ref):
        a = jnp.concatenate([a_ref[0], a_ref[1]], axis=1)
        d = dis_ref[...]
        z = jnp.maximum(a * d + b_ref[...], 0.0)
        h2 = jnp.dot(z, w_ref[...], preferred_element_type=f32)
        o_ref[...] = h2 * d

    return pl.pallas_call(
        body,
        grid=(N // BR,),
        in_specs=[
            pl.BlockSpec((2, BR, DH), lambda i: (0, i, 0)),
            pl.BlockSpec((BR, 1), lambda i: (i, 0)),
            pl.BlockSpec((1, D_HID), lambda i: (0, 0)),
            pl.BlockSpec((D_HID, D_OUT), lambda i: (0, 0)),
        ],
        out_specs=pl.BlockSpec((BR, D_OUT), lambda i: (i, 0)),
        out_shape=jax.ShapeDtypeStruct((N, D_OUT), f32),
    )(agg1, dis, b1, W2)


def _tc3(agg2, dis, b2):
    def body(a_ref, dis_ref, b_ref, o_ref):
        a = a_ref[0] + a_ref[1]
        o_ref[...] = jnp.maximum(a * dis_ref[...] + b_ref[...], 0.0)

    return pl.pallas_call(
        body,
        grid=(N // BR,),
        in_specs=[
            pl.BlockSpec((2, BR, DH), lambda i: (0, i, 0)),
            pl.BlockSpec((BR, 1), lambda i: (i, 0)),
            pl.BlockSpec((1, D_OUT), lambda i: (0, 0)),
        ],
        out_specs=pl.BlockSpec((BR, D_OUT), lambda i: (i, 0)),
        out_shape=jax.ShapeDtypeStruct((N, D_OUT), f32),
    )(agg2, dis, b2)


def kernel(x, edge_index, W1, b1, W2, b2):
    src = edge_index[0]
    dst = edge_index[1]
    pad_i = jnp.arange(E_PAD, dtype=jnp.int32)
    pad_src = pad_i % N
    pad_dst = N + pad_i % JUNK
    src2 = jnp.concatenate([src, pad_src]).reshape(E_ROWS, CH)
    dst2 = jnp.concatenate([dst, pad_dst]).reshape(E_ROWS, CH)
    dst32 = dst.reshape(NW, E // NW)

    hists = _sc_hist(dst32)
    histT = jnp.transpose(hists)

    g1, dis = _tc1(x, W1, histT)
    agg1 = _sc_agg(g1, src2, dst2, feature_split=True)
    g2 = _tc2(agg1, dis, b1.reshape(1, D_HID), W2)
    agg2 = _sc_agg(g2, src2, dst2, feature_split=False)
    out = _tc3(agg2, dis, b2.reshape(1, D_OUT))
    return out


# speedup vs baseline: 2.6444x; 1.0805x over previous
import dataclasses
import functools

import jax
import jax.numpy as jnp
from jax import lax
from jax.experimental import pallas as pl
from jax.experimental.pallas import tpu as pltpu
from jax.experimental.pallas import tpu_sc as plsc

N = 10000
E = 320000
D_IN = 128
D_HID = 256
D_OUT = 128
DH = 128

NC = 2
NS = 16
NW = NC * NS

CH = 64
E_ROWS = 5120
E_PAD = E_ROWS * CH - E
T1 = E_ROWS // NS
T2 = E_ROWS // NW
NBUF = 4
G = 16

RB = 40
NBLK = N // RB
IBL = 16
JUNK = 512
N_P = N + JUNK

BR = 1000
f32 = jnp.float32


def _mesh():
    return plsc.VectorSubcoreMesh(core_axis_name="c", subcore_axis_name="s")


def _sc_params():
    cp = pltpu.CompilerParams()
    if "needs_layout_passes" in pltpu.CompilerParams.__dataclass_fields__:
        cp = dataclasses.replace(cp, needs_layout_passes=False)
    return cp


def _sc_hist(dst32):
    @functools.partial(
        pl.kernel,
        out_type=jax.ShapeDtypeStruct((NW, N), f32),
        mesh=_mesh(),
        scratch_types=[
            pltpu.VMEM((N,), f32),
            pltpu.VMEM((E // NW,), jnp.int32),
            pltpu.SemaphoreType.DMA,
        ],
        compiler_params=_sc_params(),
    )
    def k(dst_hbm, out_hbm, hist_v, idx_v, sem):
        c = lax.axis_index("c")
        s = lax.axis_index("s")
        wid = s * NC + c
        zero16 = jnp.zeros((16,), f32)
        one16 = jnp.full((16,), 1.0, f32)

        cp = pltpu.async_copy(dst_hbm.at[wid], idx_v, sem)

        @pl.loop(0, N // 16)
        def _(i):
            hist_v[pl.ds(i * 16, 16)] = zero16

        cp.wait()

        @pl.loop(0, (E // NW) // 16)
        def _(i):
            idx = idx_v[pl.ds(i * 16, 16)]
            plsc.addupdate_scatter(hist_v, [idx], one16)

        pltpu.sync_copy(hist_v, out_hbm.at[wid])

    return k(dst32)


def _sc_agg(g, src2, dst2, feature_split):
    TCNT = T1 if feature_split else T2
    TG = TCNT // G

    @functools.partial(
        pl.kernel,
        out_type=jax.ShapeDtypeStruct((NC, N_P, DH), f32),
        mesh=_mesh(),
        scratch_types=[
            pltpu.VMEM_SHARED((N_P, DH), f32),
            pltpu.VMEM((G, CH), jnp.int32),
            pltpu.VMEM((G, CH), jnp.int32),
            pltpu.VMEM((NBUF, CH, DH), f32),
            [pltpu.SemaphoreType.DMA] * NBUF,
            [pltpu.SemaphoreType.DMA] * NBUF,
        ],
        compiler_params=_sc_params(),
    )
    def k(g_hbm, src_hbm, dst_hbm, out_hbm, acc_sh, src_idx, dst_idx,
          rows_v, gsem, ssem):
        c = lax.axis_index("c")
        s = lax.axis_index("s")

        gsrc = g_hbm.at[c] if feature_split else g_hbm
        lo = s * TCNT if feature_split else (c * NS + s) * TCNT

        if not feature_split:
            zero16 = jnp.zeros((16,), f32)

            @pl.when(c == 1)
            def _():
                @pl.loop(0, RB)
                def _(r):
                    for j in range(DH // 16):
                        rows_v[0, r, pl.ds(j * 16, 16)] = zero16

        for b in range(IBL):
            kk = b * NS + s
            base = kk * RB

            @pl.when(kk < NBLK)
            def _():
                if feature_split:
                    pltpu.sync_copy(g_hbm.at[c].at[pl.ds(base, RB)],
                                    rows_v.at[1].at[pl.ds(0, RB)])
                    pltpu.sync_copy(rows_v.at[1].at[pl.ds(0, RB)],
                                    acc_sh.at[pl.ds(base, RB)])
                else:
                    @pl.when(c == 0)
                    def _():
                        pltpu.sync_copy(g_hbm.at[pl.ds(base, RB)],
                                        rows_v.at[1].at[pl.ds(0, RB)])
                        pltpu.sync_copy(rows_v.at[1].at[pl.ds(0, RB)],
                                        acc_sh.at[pl.ds(base, RB)])

                    @pl.when(c == 1)
                    def _():
                        pltpu.sync_copy(rows_v.at[0].at[pl.ds(0, RB)],
                                        acc_sh.at[pl.ds(base, RB)])

        plsc.subcore_barrier()

        def issue_gather(b, j):
            pltpu.async_copy(gsrc.at[src_idx.at[j]], rows_v.at[b], gsem[b])

        def wait_gather(b):
            pltpu.make_async_copy(gsrc.at[src_idx.at[0]], rows_v.at[b],
                                  gsem[b]).wait()

        def issue_scatter(b, j):
            pltpu.async_copy(rows_v.at[b], acc_sh.at[dst_idx.at[j]],
                             ssem[b], add=True)

        def wait_scatter(b):
            pltpu.make_async_copy(rows_v.at[b], acc_sh.at[dst_idx.at[0]],
                                  ssem[b]).wait()

        @pl.loop(0, TG)
        def _(gg):
            row0 = lo + gg * G
            pltpu.sync_copy(src_hbm.at[pl.ds(row0, G)], src_idx)
            pltpu.sync_copy(dst_hbm.at[pl.ds(row0, G)], dst_idx)
            for b in range(NBUF):
                issue_gather(b, b)

            @pl.loop(0, G // NBUF - 1)
            def _(r):
                for b in range(NBUF):
                    wait_gather(b)
                    issue_scatter(b, r * NBUF + b)
                for b in range(NBUF):
                    wait_scatter(b)
                    issue_gather(b, (r + 1) * NBUF + b)

            for b in range(NBUF):
                wait_gather(b)
                issue_scatter(b, G - NBUF + b)
            for b in range(NBUF):
                wait_scatter(b)

        plsc.subcore_barrier()

        for b in range(IBL):
            kk = b * NS + s
            base = kk * RB

            @pl.when(kk < NBLK)
            def _():
                pltpu.sync_copy(acc_sh.at[pl.ds(base, RB)],
                                rows_v.at[0].at[pl.ds(0, RB)])
                pltpu.sync_copy(rows_v.at[0].at[pl.ds(0, RB)],
                                out_hbm.at[c].at[pl.ds(base, RB)])

    return k(g, src2, dst2)


def _tc1(x, W1, histT):
    def body(x_ref, w_ref, h_ref, g_ref, dis_ref):
        cnt = jnp.sum(h_ref[...], axis=1, keepdims=True)
        dis = lax.rsqrt(cnt + 1.0)
        h1 = jnp.dot(x_ref[...], w_ref[...], preferred_element_type=f32)
        gg = h1 * dis
        g_ref[0] = gg[:, :DH]
        g_ref[1] = gg[:, DH:]
        dis_ref[...] = dis

    return pl.pallas_call(
        body,
        grid=(N // BR,),
        in_specs=[
            pl.BlockSpec((BR, D_IN), lambda i: (i, 0)),
            pl.BlockSpec((D_IN, D_HID), lambda i: (0, 0)),
            pl.BlockSpec((BR, NW), lambda i: (i, 0)),
        ],
        out_specs=[
            pl.BlockSpec((2, BR, DH), lambda i: (0, i, 0)),
            pl.BlockSpec((BR, 1), lambda i: (i, 0)),
        ],
        out_shape=[
            jax.ShapeDtypeStruct((NC, N, DH), f32),
            jax.ShapeDtypeStruct((N, 1), f32),
        ],
    )(x, W1, histT)


def _tc2(agg1, dis, b1, W2):
    def body(a_ref, dis_ref, b_ref, w_ref, o_ref):
        a = jnp.concatenate([a_ref[0], a_ref[1]], axis=1)
        d = dis_ref[...]
        z = jnp.maximum(a * d + b_ref[...], 0.0)
        h2 = jnp.dot(z, w_ref[...], preferred_element_type=f32)
        o_ref[...] = h2 * d

    return pl.pallas_call(
        body,
        grid=(N // BR,),
        in_specs=[
            pl.BlockSpec((2, BR, DH), lambda i: (0, i, 0)),
            pl.BlockSpec((BR, 1), lambda i: (i, 0)),
            pl.BlockSpec((1, D_HID), lambda i: (0, 0)),
            pl.BlockSpec((D_HID, D_OUT), lambda i: (0, 0)),
        ],
        out_specs=pl.BlockSpec((BR, D_OUT), lambda i: (i, 0)),
        out_shape=jax.ShapeDtypeStruct((N, D_OUT), f32),
    )(agg1, dis, b1, W2)


def _tc3(agg2, dis, b2):
    def body(a_ref, dis_ref, b_ref, o_ref):
        a = a_ref[0] + a_ref[1]
        o_ref[...] = jnp.maximum(a * dis_ref[...] + b_ref[...], 0.0)

    return pl.pallas_call(
        body,
        grid=(N // BR,),
        in_specs=[
            pl.BlockSpec((2, BR, DH), lambda i: (0, i, 0)),
            pl.BlockSpec((BR, 1), lambda i: (i, 0)),
            pl.BlockSpec((1, D_OUT), lambda i: (0, 0)),
        ],
        out_specs=pl.BlockSpec((BR, D_OUT), lambda i: (i, 0)),
        out_shape=jax.ShapeDtypeStruct((N, D_OUT), f32),
    )(agg2, dis, b2)


def kernel(x, edge_index, W1, b1, W2, b2):
    src = edge_index[0]
    dst = edge_index[1]
    pad_i = jnp.arange(E_PAD, dtype=jnp.int32)
    pad_src = pad_i % N
    pad_dst = N + pad_i % JUNK
    src2 = jnp.concatenate([src, pad_src]).reshape(E_ROWS, CH)
    dst2 = jnp.concatenate([dst, pad_dst]).reshape(E_ROWS, CH)
    dst32 = dst.reshape(NW, E // NW)

    hists = _sc_hist(dst32)
    histT = jnp.transpose(hists)

    g1, dis = _tc1(x, W1, histT)
    agg1 = _sc_agg(g1, src2, dst2, feature_split=True)
    g2 = _tc2(agg1, dis, b1.reshape(1, D_HID), W2)
    agg2 = _sc_agg(g2, src2, dst2, feature_split=False)
    out = _tc3(agg2, dis, b2.reshape(1, D_OUT))
    return out
